# baseline (speedup 1.0000x reference)
import functools

import jax
import jax.numpy as jnp
from jax import lax
from jax.experimental import pallas as pl
from jax.experimental.pallas import tpu as pltpu
from jax.experimental.pallas import tpu_sc as plsc

_NC = 2
_NS = 16
_NW = _NC * _NS
_CHUNK = 32
_NBUF = 4


def _make_sc_broadcast(bsz, max_len, d_model):
    rows_per_w = max_len // _NW
    n_chunks = rows_per_w // _CHUNK

    @functools.partial(
        pl.kernel,
        out_type=jax.ShapeDtypeStruct((bsz, max_len, d_model), jnp.float32),
        mesh=plsc.VectorSubcoreMesh(core_axis_name="c", subcore_axis_name="s"),
        scratch_types=(
            [pltpu.VMEM((_CHUNK, d_model), jnp.float32)] * _NBUF
            + [pltpu.SemaphoreType.DMA] * (2 * _NBUF)
        ),
    )
    def k(table_hbm, out_hbm, *rest):
        bufs = rest[:_NBUF]
        sins = rest[_NBUF:2 * _NBUF]
        souts = rest[2 * _NBUF:]
        wid = lax.axis_index("s") * _NC + lax.axis_index("c")
        base = wid * rows_per_w

        def in_desc(i):
            return pltpu.make_async_copy(
                table_hbm.at[pl.ds(base + i * _CHUNK, _CHUNK), :],
                bufs[i % _NBUF], sins[i % _NBUF])

        def out_desc(i, b):
            return pltpu.make_async_copy(
                bufs[i % _NBUF],
                out_hbm.at[b, pl.ds(base + i * _CHUNK, _CHUNK), :],
                souts[i % _NBUF])

        for i in range(min(_NBUF - 1, n_chunks)):
            in_desc(i).start()
        outs_waited = -1
        for i in range(n_chunks):
            in_desc(i).wait()
            for b in range(bsz):
                out_desc(i, b).start()
            j = i + _NBUF - 1
            if j < n_chunks:
                if i >= 1:
                    for b in range(bsz):
                        out_desc(i - 1, b).wait()
                    outs_waited = i - 1
                in_desc(j).start()
        for i in range(outs_waited + 1, n_chunks):
            for b in range(bsz):
                out_desc(i, b).wait()

    return k


def kernel(x, pos_embed_weight):
    bsz, seq_len = x.shape
    max_len, d_model = pos_embed_weight.shape
    return _make_sc_broadcast(bsz, max_len, d_model)(pos_embed_weight)


# speedup vs baseline: 1.0293x; 1.0293x over previous
import functools

import jax
import jax.numpy as jnp
from jax import lax
from jax.experimental import pallas as pl
from jax.experimental.pallas import tpu as pltpu
from jax.experimental.pallas import tpu_sc as plsc

_NC = 2
_NS = 16
_NW = _NC * _NS
_CHUNK = 128
_NBUF = 1


def _make_sc_broadcast(bsz, max_len, d_model):
    rows_per_w = max_len // _NW
    n_chunks = rows_per_w // _CHUNK

    @functools.partial(
        pl.kernel,
        out_type=jax.ShapeDtypeStruct((bsz, max_len, d_model), jnp.float32),
        mesh=plsc.VectorSubcoreMesh(core_axis_name="c", subcore_axis_name="s"),
        scratch_types=(
            [pltpu.VMEM((_CHUNK, d_model), jnp.float32)] * _NBUF
            + [pltpu.SemaphoreType.DMA] * (2 * _NBUF)
        ),
    )
    def k(table_hbm, out_hbm, *rest):
        bufs = rest[:_NBUF]
        sins = rest[_NBUF:2 * _NBUF]
        souts = rest[2 * _NBUF:]
        wid = lax.axis_index("s") * _NC + lax.axis_index("c")
        base = wid * rows_per_w

        def in_desc(i):
            return pltpu.make_async_copy(
                table_hbm.at[pl.ds(base + i * _CHUNK, _CHUNK), :],
                bufs[i % _NBUF], sins[i % _NBUF])

        def out_desc(i, b):
            return pltpu.make_async_copy(
                bufs[i % _NBUF],
                out_hbm.at[b, pl.ds(base + i * _CHUNK, _CHUNK), :],
                souts[i % _NBUF])

        if _NBUF == 1:
            for i in range(n_chunks):
                in_desc(i).start()
                in_desc(i).wait()
                for b in range(bsz):
                    out_desc(i, b).start()
                for b in range(bsz):
                    out_desc(i, b).wait()
        else:
            for i in range(min(_NBUF - 1, n_chunks)):
                in_desc(i).start()
            outs_waited = -1
            for i in range(n_chunks):
                in_desc(i).wait()
                for b in range(bsz):
                    out_desc(i, b).start()
                j = i + _NBUF - 1
                if j < n_chunks:
                    if i >= 1:
                        for b in range(bsz):
                            out_desc(i - 1, b).wait()
                        outs_waited = i - 1
                    in_desc(j).start()
            for i in range(outs_waited + 1, n_chunks):
                for b in range(bsz):
                    out_desc(i, b).wait()

    return k


def kernel(x, pos_embed_weight):
    bsz, seq_len = x.shape
    max_len, d_model = pos_embed_weight.shape
    return _make_sc_broadcast(bsz, max_len, d_model)(pos_embed_weight)
